# baseline (speedup 1.0000x reference)
import jax
import jax.numpy as jnp
from jax.experimental import pallas as pl
from jax.experimental.pallas import tpu as pltpu

_BM = 2048


def _rbf_kernel(sc_ref, xt_ref, z_ref, xg_ref, zg_ref, out_ref):
    q0 = sc_ref[0]
    q1 = sc_ref[1]
    xg = xg_ref[0]
    a0 = xg[:, 0:1] * q0
    a1 = xg[:, 1:2] * q1
    b0 = xt_ref[0, 0:1, :] * q0
    b1 = xt_ref[0, 1:2, :] * q1
    d0 = a0 - b0
    d1 = a1 - b1
    w = jnp.exp2(-(d0 * d0 + d1 * d1)).astype(jnp.bfloat16)
    acc = jax.lax.dot_general(
        w, z_ref[0], (((1,), (0,)), ((), ())),
        preferred_element_type=jnp.float32,
        precision=jax.lax.Precision.DEFAULT)
    out_ref[0] = zg_ref[0] + acc


@jax.jit
def kernel(x, z, x_grid, z_grid, lengthscale_param):
    m, n, dx = x.shape
    dz = z.shape[-1]
    grid_spatial = x_grid.shape[1:-1]
    M = 1
    for s in grid_spatial:
        M *= s

    lengthscale = 1e-5 + jax.nn.softplus(lengthscale_param)
    sc = (jnp.sqrt(jnp.log2(jnp.e) * 0.5) / lengthscale).astype(jnp.float32)

    xt = jnp.swapaxes(x, 1, 2)
    xg_flat = x_grid.reshape(m, M, dx)
    zg_flat = z_grid.reshape(m, M, dz)

    grid = (m, M // _BM)
    out = pl.pallas_call(
        _rbf_kernel,
        grid=grid,
        in_specs=[
            pl.BlockSpec(memory_space=pltpu.SMEM),
            pl.BlockSpec((1, dx, n), lambda b, i: (b, 0, 0)),
            pl.BlockSpec((1, n, dz), lambda b, i: (b, 0, 0)),
            pl.BlockSpec((1, _BM, dx), lambda b, i: (b, i, 0)),
            pl.BlockSpec((1, _BM, dz), lambda b, i: (b, i, 0)),
        ],
        out_specs=pl.BlockSpec((1, _BM, dz), lambda b, i: (b, i, 0)),
        out_shape=jax.ShapeDtypeStruct((m, M, dz), jnp.float32),
        compiler_params=pltpu.CompilerParams(
            dimension_semantics=("parallel", "parallel")),
    )(sc, xt, z.astype(jnp.bfloat16), xg_flat, zg_flat)

    return (x_grid, out.reshape(z_grid.shape))


# speedup vs baseline: 1.0161x; 1.0161x over previous
import jax
import jax.numpy as jnp
from jax.experimental import pallas as pl
from jax.experimental.pallas import tpu as pltpu

_BM = 2048


def _rbf_kernel(sc_ref, xt_ref, x_ref, z_ref, xg_ref, zg_ref, out_ref):
    q0 = sc_ref[0]
    q1 = sc_ref[1]
    xg = xg_ref[0]
    u0 = (xg[:, 0:1] - 0.5) * q0
    u1 = (xg[:, 1:2] - 0.5) * q1
    row_a = jnp.exp2(-(u0 * u0 + u1 * u1))
    a0 = u0 + u0
    a1 = u1 + u1
    v0 = (xt_ref[0, 0:1, :] - 0.5) * q0
    v1 = (xt_ref[0, 1:2, :] - 0.5) * q1
    w0 = (x_ref[0, :, 0:1] - 0.5) * q0
    w1 = (x_ref[0, :, 1:2] - 0.5) * q1
    col_b = jnp.exp2(-(w0 * w0 + w1 * w1))
    zs = z_ref[0] * col_b
    c = a0 * v0 + a1 * v1
    w = jnp.exp2(c)
    p = jnp.dot(w, zs, preferred_element_type=jnp.float32)
    out_ref[0] = zg_ref[0] + row_a * p


@jax.jit
def kernel(x, z, x_grid, z_grid, lengthscale_param):
    m, n, dx = x.shape
    dz = z.shape[-1]
    grid_spatial = x_grid.shape[1:-1]
    M = 1
    for s in grid_spatial:
        M *= s

    lengthscale = 1e-5 + jax.nn.softplus(lengthscale_param)
    sc = (jnp.sqrt(jnp.log2(jnp.e) * 0.5) / lengthscale).astype(jnp.float32)

    xt = jnp.swapaxes(x, 1, 2)
    xg_flat = x_grid.reshape(m, M, dx)
    zg_flat = z_grid.reshape(m, M, dz)

    grid = (m, M // _BM)
    out = pl.pallas_call(
        _rbf_kernel,
        grid=grid,
        in_specs=[
            pl.BlockSpec(memory_space=pltpu.SMEM),
            pl.BlockSpec((1, dx, n), lambda b, i: (b, 0, 0)),
            pl.BlockSpec((1, n, dx), lambda b, i: (b, 0, 0)),
            pl.BlockSpec((1, n, dz), lambda b, i: (b, 0, 0)),
            pl.BlockSpec((1, _BM, dx), lambda b, i: (b, i, 0)),
            pl.BlockSpec((1, _BM, dz), lambda b, i: (b, i, 0)),
        ],
        out_specs=pl.BlockSpec((1, _BM, dz), lambda b, i: (b, i, 0)),
        out_shape=jax.ShapeDtypeStruct((m, M, dz), jnp.float32),
        compiler_params=pltpu.CompilerParams(
            dimension_semantics=("parallel", "parallel")),
    )(sc, xt, x, z, xg_flat, zg_flat)

    return (x_grid, out.reshape(z_grid.shape))
